# Initial kernel scaffold; baseline (speedup 1.0000x reference)
import jax
import jax.numpy as jnp
from jax.experimental import pallas as pl


def kernel(xyz, params):
    raise NotImplementedError("write your pallas kernel here")


# speedup vs baseline: 7.3677x; 7.3677x over previous
import jax
import jax.numpy as jnp
from jax.experimental import pallas as pl
from jax.experimental.pallas import tpu as pltpu

F32 = jnp.float32
BIG = 3.0e38


def _mm(x, w):
    return jnp.dot(x, w, preferred_element_type=F32)


def _first_min_mask(d, iota, n):
    dmin = jnp.min(d, axis=1, keepdims=True)
    first = jnp.min(jnp.where(d == dmin, iota, n), axis=1, keepdims=True)
    return iota == first, dmin


def _density(xyz, xyzT, bw, tq):
    B, n, _ = xyz.shape
    inv2bw2 = 1.0 / (2.0 * bw * bw)
    coef = 1.0 / (2.5 * bw)

    def kfn(q_ref, kT_ref, o_ref):
        q = q_ref[...]
        kT = kT_ref[...]
        qq = jnp.sum(q * q, axis=1, keepdims=True)
        kk = jnp.sum(kT * kT, axis=0, keepdims=True)
        d = jnp.maximum(qq + kk - 2.0 * _mm(q, kT), 0.0)
        g = jnp.exp(d * (-inv2bw2)) * coef
        o_ref[...] = 1.0 / jnp.mean(g, axis=1, keepdims=True)

    return pl.pallas_call(
        kfn,
        grid=(B, n // tq),
        in_specs=[
            pl.BlockSpec((None, tq, 3), lambda b, i: (b, i, 0)),
            pl.BlockSpec((None, 3, n), lambda b, i: (b, 0, 0)),
        ],
        out_specs=pl.BlockSpec((None, tq, 1), lambda b, i: (b, i, 0)),
        out_shape=jax.ShapeDtypeStruct((B, n, 1), F32),
    )(xyz, xyzT)


def _fps(xyz, xyzT, m):
    B, n, _ = xyz.shape

    def kfn(x_ref, xT_ref, o_ref):
        xT = xT_ref[...]
        kk = jnp.sum(xT * xT, axis=0, keepdims=True)
        row0 = x_ref[0:1, :]
        rr0 = jnp.sum(row0 * row0, keepdims=True)
        dist = kk - 2.0 * _mm(row0, xT) + rr0
        o_ref[0:1, :] = row0

        def body(i, dist):
            nxt = jnp.argmax(dist)
            row = x_ref[pl.ds(nxt, 1), :]
            o_ref[pl.ds(i, 1), :] = row
            rr = jnp.sum(row * row, keepdims=True)
            d = kk - 2.0 * _mm(row, xT) + rr
            return jnp.minimum(dist, d)

        jax.lax.fori_loop(1, m, body, dist)

    return pl.pallas_call(
        kfn,
        grid=(B,),
        in_specs=[
            pl.BlockSpec((None, n, 3), lambda b: (b, 0, 0)),
            pl.BlockSpec((None, 3, n), lambda b: (b, 0, 0)),
        ],
        out_specs=pl.BlockSpec((None, m, 3), lambda b: (b, 0, 0)),
        out_shape=jax.ShapeDtypeStruct((B, m, 3), F32),
    )(xyz, xyzT)


NS = 32


def _sa(new_xyz, xyzT, ext, p, tqm):
    B, m, _ = new_xyz.shape
    n = xyzT.shape[2]
    C = ext.shape[2] - 4
    (W1, b1), (W2, b2), (W3, b3) = p['mlp']
    (V1, c1), (V2, c2), (V3, c3) = p['weightnet']
    (D1, e1), (D2, e2), (D3, e3) = p['densitynet']
    Wl, bl = p['linear']
    Cm = W3.shape[1]
    CL = Wl.shape[1]
    Wlp = Wl.reshape(Cm, 16, CL).transpose(1, 0, 2).reshape(16 * Cm, CL)
    prm = [W1[:3], W1[3:], b1.reshape(1, -1), W2, b2.reshape(1, -1),
           W3, b3.reshape(1, -1),
           V1, c1.reshape(1, -1), V2, c2.reshape(1, -1), V3, c3.reshape(1, -1),
           D1, e1.reshape(1, -1), D2, e2.reshape(1, -1), D3, e3.reshape(1, -1),
           Wlp, bl.reshape(1, -1)]

    def kfn(q_ref, kT_ref, ext_ref,
            W1a_ref, W1b_ref, b1_ref, W2_ref, b2_ref, W3_ref, b3_ref,
            V1_ref, c1_ref, V2_ref, c2_ref, V3_ref, c3_ref,
            D1_ref, e1_ref, D2_ref, e2_ref, D3_ref, e3_ref,
            Wl_ref, bl_ref, o_ref, Gn_ref, Gp_ref, Gd_ref):
        q = q_ref[...]
        kT = kT_ref[...]
        ext_v = ext_ref[...]
        qq = jnp.sum(q * q, axis=1, keepdims=True)
        kk = jnp.sum(kT * kT, axis=0, keepdims=True)
        d = jnp.maximum(qq + kk - 2.0 * _mm(q, kT), 0.0)
        iota = jax.lax.broadcasted_iota(jnp.int32, (tqm, n), 1)
        dmax = None
        for j in range(NS):
            mask, _ = _first_min_mask(d, iota, n)
            feat = _mm(mask.astype(F32), ext_v)
            d = jnp.where(mask, BIG, d)
            Gn_ref[j * tqm:(j + 1) * tqm, :] = feat[:, 0:3] - q
            Gp_ref[j * tqm:(j + 1) * tqm, :] = feat[:, 3:3 + C]
            dj = feat[:, 3 + C:4 + C]
            Gd_ref[j * tqm:(j + 1) * tqm, :] = dj
            dmax = dj if j == 0 else jnp.maximum(dmax, dj)
        for j in range(NS):
            Gd_ref[j * tqm:(j + 1) * tqm, :] = (
                Gd_ref[j * tqm:(j + 1) * tqm, :] / dmax)

        Xn = Gn_ref[...]
        h = jax.nn.relu(_mm(Xn, W1a_ref[...]) + _mm(Gp_ref[...], W1b_ref[...])
                        + b1_ref[...])
        h = jax.nn.relu(_mm(h, W2_ref[...]) + b2_ref[...])
        npts = jax.nn.relu(_mm(h, W3_ref[...]) + b3_ref[...])

        s = jax.nn.relu(_mm(Gd_ref[...], D1_ref[...]) + e1_ref[...])
        s = jax.nn.relu(_mm(s, D2_ref[...]) + e2_ref[...])
        s = jax.nn.sigmoid(_mm(s, D3_ref[...]) + e3_ref[...])
        npts = npts * s

        w = jax.nn.relu(_mm(Xn, V1_ref[...]) + c1_ref[...])
        w = jax.nn.relu(_mm(w, V2_ref[...]) + c2_ref[...])
        w = jax.nn.relu(_mm(w, V3_ref[...]) + c3_ref[...])

        acc = jnp.zeros((tqm, 16, Cm), F32)
        for j in range(NS):
            npj = npts[j * tqm:(j + 1) * tqm, :]
            wj = w[j * tqm:(j + 1) * tqm, :]
            acc = acc + wj[:, :, None] * npj[:, None, :]
        out = jnp.broadcast_to(bl_ref[...], (tqm, CL))
        for wdx in range(16):
            out = out + _mm(acc[:, wdx, :], Wl_ref[wdx * Cm:(wdx + 1) * Cm, :])
        o_ref[...] = jax.nn.relu(out)

    F = C + 4
    pspecs = [pl.BlockSpec(a.shape, lambda b, i: (0, 0)) for a in prm]
    return pl.pallas_call(
        kfn,
        grid=(B, m // tqm),
        in_specs=[
            pl.BlockSpec((None, tqm, 3), lambda b, i: (b, i, 0)),
            pl.BlockSpec((None, 3, n), lambda b, i: (b, 0, 0)),
            pl.BlockSpec((None, n, F), lambda b, i: (b, 0, 0)),
        ] + pspecs,
        out_specs=pl.BlockSpec((None, tqm, CL), lambda b, i: (b, i, 0)),
        out_shape=jax.ShapeDtypeStruct((B, m, CL), F32),
        scratch_shapes=[
            pltpu.VMEM((NS * tqm, 3), F32),
            pltpu.VMEM((NS * tqm, C), F32),
            pltpu.VMEM((NS * tqm, 1), F32),
        ],
    )(new_xyz, xyzT, ext, *prm)


def _fp(xyz1, xyz2T, points1, points2, mlp_params, tq):
    B, n1, _ = xyz1.shape
    n2 = xyz2T.shape[2]
    C1 = points1.shape[2]
    C2 = points2.shape[2]
    W0, b0 = mlp_params[0]
    prm = [W0[:C1], W0[C1:], b0.reshape(1, -1)]
    for Wi, bi in mlp_params[1:]:
        prm += [Wi, bi.reshape(1, -1)]
    nrest = len(mlp_params) - 1
    CL = mlp_params[-1][0].shape[1]

    def kfn(*refs):
        q_ref, kT_ref, p2_ref, p1_ref = refs[:4]
        Wa_ref, Wb_ref, b0_ref = refs[4:7]
        rest = refs[7:7 + 2 * nrest]
        o_ref = refs[7 + 2 * nrest]
        q = q_ref[...]
        kT = kT_ref[...]
        p2 = p2_ref[...]
        qq = jnp.sum(q * q, axis=1, keepdims=True)
        kk = jnp.sum(kT * kT, axis=0, keepdims=True)
        d = jnp.maximum(qq + kk - 2.0 * _mm(q, kT), 0.0)
        iota = jax.lax.broadcasted_iota(jnp.int32, (tq, n2), 1)
        interp = jnp.zeros((tq, C2), F32)
        wsum = jnp.zeros((tq, 1), F32)
        for j in range(3):
            mask, dmin = _first_min_mask(d, iota, n2)
            feat = _mm(mask.astype(F32), p2)
            wj = 1.0 / jnp.maximum(dmin, 1e-10)
            interp = interp + feat * wj
            wsum = wsum + wj
            d = jnp.where(mask, BIG, d)
        interp = interp / wsum
        h = jax.nn.relu(_mm(p1_ref[...], Wa_ref[...]) + _mm(interp, Wb_ref[...])
                        + b0_ref[...])
        for li in range(nrest):
            h = jax.nn.relu(_mm(h, rest[2 * li][...]) + rest[2 * li + 1][...])
        o_ref[...] = h

    pspecs = [pl.BlockSpec(a.shape, lambda b, i: (0, 0)) for a in prm]
    return pl.pallas_call(
        kfn,
        grid=(B, n1 // tq),
        in_specs=[
            pl.BlockSpec((None, tq, 3), lambda b, i: (b, i, 0)),
            pl.BlockSpec((None, 3, n2), lambda b, i: (b, 0, 0)),
            pl.BlockSpec((None, n2, C2), lambda b, i: (b, 0, 0)),
            pl.BlockSpec((None, tq, C1), lambda b, i: (b, i, 0)),
        ] + pspecs,
        out_specs=pl.BlockSpec((None, tq, CL), lambda b, i: (b, i, 0)),
        out_shape=jax.ShapeDtypeStruct((B, n1, CL), F32),
    )(xyz1, xyz2T, points2, points1, *prm)


def _fc(x, params, tq):
    B, n, C = x.shape
    (W1, b1) = params['fc1']
    (W2, b2) = params['fc2']
    (W3, b3) = params['fc3']
    prm = [W1, b1.reshape(1, -1), W2, b2.reshape(1, -1), W3, b3.reshape(1, -1)]

    def kfn(x_ref, W1_ref, b1_ref, W2_ref, b2_ref, W3_ref, b3_ref, o_ref):
        h = jax.nn.relu(_mm(x_ref[...], W1_ref[...]) + b1_ref[...])
        h = jax.nn.relu(_mm(h, W2_ref[...]) + b2_ref[...])
        t = _mm(h, W3_ref[...]) + b3_ref[...]
        s = t - jnp.max(t, axis=1, keepdims=True)
        o_ref[...] = s - jnp.log(jnp.sum(jnp.exp(s), axis=1, keepdims=True))

    pspecs = [pl.BlockSpec(a.shape, lambda b, i: (0, 0)) for a in prm]
    return pl.pallas_call(
        kfn,
        grid=(B, n // tq),
        in_specs=[pl.BlockSpec((None, tq, C), lambda b, i: (b, i, 0))] + pspecs,
        out_specs=pl.BlockSpec((None, tq, 1), lambda b, i: (b, i, 0)),
        out_shape=jax.ShapeDtypeStruct((B, n, 1), F32),
    )(x, *prm)


def _sa_level(xyz_l, points_l, p, m, bw, tq_d, tqm):
    xyzT = jnp.transpose(xyz_l, (0, 2, 1))
    invd = _density(xyz_l, xyzT, bw, tq_d)
    new_xyz = _fps(xyz_l, xyzT, m)
    ext = jnp.concatenate([xyz_l, points_l, invd], axis=2)
    new_points = _sa(new_xyz, xyzT, ext, p, tqm)
    return new_xyz, new_points


def _forward(xyz, params):
    l0_xyz = jnp.transpose(xyz, (0, 2, 1))
    l0_points = l0_xyz
    l1_xyz, l1_points = _sa_level(l0_xyz, l0_points, params['sa1'],
                                  1024, 0.1, 512, 256)
    l2_xyz, l2_points = _sa_level(l1_xyz, l1_points, params['sa2'],
                                  256, 0.2, 512, 256)
    l3_xyz, l3_points = _sa_level(l2_xyz, l2_points, params['sa3'],
                                  64, 0.4, 256, 64)
    l4_xyz, l4_points = _sa_level(l3_xyz, l3_points, params['sa4'],
                                  36, 0.4, 64, 36)
    l4T = jnp.transpose(l4_xyz, (0, 2, 1))
    l3T = jnp.transpose(l3_xyz, (0, 2, 1))
    l2T = jnp.transpose(l2_xyz, (0, 2, 1))
    l1T = jnp.transpose(l1_xyz, (0, 2, 1))
    l3_points = _fp(l3_xyz, l4T, l3_points, l4_points, params['fp1'], 64)
    l2_points = _fp(l2_xyz, l3T, l2_points, l3_points, params['fp2'], 256)
    l1_points = _fp(l1_xyz, l2T, l1_points, l2_points, params['fp3'], 512)
    pts = _fp(l0_xyz, l1T, l0_points, l1_points, params['fp4'], 512)
    return _fc(pts, params, 1024)


def kernel(xyz, params):
    return _forward(xyz, params)
